# Initial kernel scaffold; baseline (speedup 1.0000x reference)
import jax
import jax.numpy as jnp
from jax.experimental import pallas as pl


def kernel(x, input_idx, input_weights, process_indices, input_neurons, process_neurons):
    raise NotImplementedError("write your pallas kernel here")


# speedup vs baseline: 3.1155x; 3.1155x over previous
import functools

import jax
import jax.numpy as jnp
from jax.experimental import pallas as pl

B, S, D, R, N_INPUT, N_PROCESS, K = 4, 2048, 2048, 256, 8, 32, 8
TOK_BLK = 256


def _fused_body(x_ref, w_ref, iw_ref, idx_ref, tab_ref, out_ref):
    xb = x_ref[...].astype(jnp.bfloat16)
    iw = iw_ref[...]
    h = jnp.zeros((TOK_BLK, R), dtype=jnp.float32)
    for n in range(N_INPUT):
        p = jax.lax.dot_general(
            xb, w_ref[n],
            (((1,), (0,)), ((), ())),
            preferred_element_type=jnp.float32,
        )
        wn = jax.lax.slice(iw, (0, n), (TOK_BLK, n + 1))
        h = h + wn * p

    t = tab_ref[...]
    tn = t / jnp.sqrt(jnp.sum(t * t, axis=1, keepdims=True) + 1e-8)

    idx = idx_ref[...]
    for k in range(K):
        ik = jax.lax.slice(idx, (0, k), (TOK_BLK, k + 1))
        oh = (jax.lax.broadcasted_iota(jnp.int32, (TOK_BLK, N_PROCESS), 1)
              == ik).astype(jnp.float32)
        vn = jax.lax.dot_general(
            oh, tn,
            (((1,), (0,)), ((), ())),
            preferred_element_type=jnp.float32,
            precision=jax.lax.Precision.HIGHEST,
        )
        d = jnp.sum(h * vn, axis=1, keepdims=True)
        h = h - 2.0 * vn * d
    out_ref[...] = h


@functools.partial(jax.jit, static_argnames=("interpret",))
def _run(x2, iw2, idx2, w_bf, table, interpret=False):
    n_blocks = (B * S) // TOK_BLK
    return pl.pallas_call(
        _fused_body,
        grid=(n_blocks,),
        in_specs=[
            pl.BlockSpec((TOK_BLK, D), lambda i: (i, 0)),
            pl.BlockSpec((N_INPUT, D, R), lambda i: (0, 0, 0)),
            pl.BlockSpec((TOK_BLK, N_INPUT), lambda i: (i, 0)),
            pl.BlockSpec((TOK_BLK, K), lambda i: (i, 0)),
            pl.BlockSpec((N_PROCESS, R), lambda i: (0, 0)),
        ],
        out_specs=pl.BlockSpec((TOK_BLK, R), lambda i: (i, 0)),
        out_shape=jax.ShapeDtypeStruct((B * S, R), jnp.float32),
        interpret=interpret,
    )(x2, w_bf, iw2, idx2, table)


def kernel(x, input_idx, input_weights, process_indices, input_neurons,
           process_neurons, *, interpret=False):
    del input_idx
    x2 = x.reshape(B * S, D)
    iw2 = input_weights.reshape(B * S, N_INPUT)
    idx2 = process_indices.reshape(B * S, K).astype(jnp.int32)
    w_bf = input_neurons.astype(jnp.bfloat16)
    h = _run(x2, iw2, idx2, w_bf, process_neurons, interpret=interpret)
    return h.reshape(B, S, R)


# speedup vs baseline: 4.3719x; 1.4033x over previous
import functools

import jax
import jax.numpy as jnp
from jax.experimental import pallas as pl

B, S, D, R, N_INPUT, N_PROCESS, K = 4, 2048, 2048, 256, 8, 32, 8
TOK_BLK = 512


def _fused_body(x_ref, w_ref, iw_ref, idx_ref, tab_ref, out_ref):
    xb = x_ref[...].astype(jnp.bfloat16)
    iw = iw_ref[...]
    h = jnp.zeros((TOK_BLK, R), dtype=jnp.float32)
    for n in range(N_INPUT):
        p = jax.lax.dot_general(
            xb, w_ref[n],
            (((1,), (0,)), ((), ())),
            preferred_element_type=jnp.float32,
        )
        wn = jax.lax.slice(iw, (0, n), (TOK_BLK, n + 1))
        h = h + wn * p

    t = tab_ref[...]
    tn = t / jnp.sqrt(jnp.sum(t * t, axis=1, keepdims=True) + 1e-8)
    tn_hi = tn.astype(jnp.bfloat16)
    tn_lo = (tn - tn_hi.astype(jnp.float32)).astype(jnp.bfloat16)
    tn_cat = jnp.concatenate([tn_hi, tn_lo], axis=0)

    idx = idx_ref[...]
    lanes = jax.lax.broadcasted_iota(
        jnp.int32, (TOK_BLK, 2 * N_PROCESS), 1) & (N_PROCESS - 1)
    vns = []
    for k in range(K):
        ik = jax.lax.slice(idx, (0, k), (TOK_BLK, k + 1))
        oh = (lanes == ik).astype(jnp.bfloat16)
        vns.append(jax.lax.dot_general(
            oh, tn_cat,
            (((1,), (0,)), ((), ())),
            preferred_element_type=jnp.float32,
        ))
    for k in range(K):
        vn = vns[k]
        d = jnp.sum(h * vn, axis=1, keepdims=True)
        h = h - 2.0 * vn * d
    out_ref[...] = h


@functools.partial(jax.jit, static_argnames=("interpret",))
def _run(x2, iw2, idx2, w_bf, table, interpret=False):
    n_blocks = (B * S) // TOK_BLK
    return pl.pallas_call(
        _fused_body,
        grid=(n_blocks,),
        in_specs=[
            pl.BlockSpec((TOK_BLK, D), lambda i: (i, 0)),
            pl.BlockSpec((N_INPUT, D, R), lambda i: (0, 0, 0)),
            pl.BlockSpec((TOK_BLK, N_INPUT), lambda i: (i, 0)),
            pl.BlockSpec((TOK_BLK, K), lambda i: (i, 0)),
            pl.BlockSpec((N_PROCESS, R), lambda i: (0, 0)),
        ],
        out_specs=pl.BlockSpec((TOK_BLK, R), lambda i: (i, 0)),
        out_shape=jax.ShapeDtypeStruct((B * S, R), jnp.float32),
        interpret=interpret,
    )(x2, w_bf, iw2, idx2, table)


def kernel(x, input_idx, input_weights, process_indices, input_neurons,
           process_neurons, *, interpret=False):
    del input_idx
    x2 = x.reshape(B * S, D)
    iw2 = input_weights.reshape(B * S, N_INPUT)
    idx2 = process_indices.reshape(B * S, K).astype(jnp.int32)
    w_bf = input_neurons.astype(jnp.bfloat16)
    h = _run(x2, iw2, idx2, w_bf, process_neurons, interpret=interpret)
    return h.reshape(B, S, R)
